# baseline (speedup 1.0000x reference)
import jax
import jax.numpy as jnp
from jax import lax
from jax.experimental import pallas as pl
from jax.experimental.pallas import tpu as pltpu
from jax.experimental.pallas import tpu_sc as plsc

N_NODES = 10000
N_EDGES = 320000
NC = 2
NS = 16
NW = NC * NS
LANES = 16
TILE = 128
N_TILES = N_EDGES // TILE
E_MAX = (N_TILES // NW + 1) * TILE
PADN = 10240


def _matmul_body(x_ref, w1t_ref, w2t_ref, y_ref):
    x = x_ref[...]
    y_ref[0:1, pl.ds(0, N_NODES)] = lax.dot_general(
        w1t_ref[...], x, dimension_numbers=(((1,), (1,)), ((), ())),
        preferred_element_type=jnp.float32)
    y_ref[0:1, pl.ds(PADN, N_NODES)] = lax.dot_general(
        w2t_ref[...], x, dimension_numbers=(((1,), (1,)), ((), ())),
        preferred_element_type=jnp.float32)


def _node_tables(X, W1, W2):
    return pl.pallas_call(
        _matmul_body,
        out_shape=jax.ShapeDtypeStruct((1, 2 * PADN), jnp.float32),
    )(X, W1.T, W2.T)


def _edge_body(y_hbm, ei_hbm, out_hbm, y_sh, y_v, ei_v, out_v,
               sem_y, sem_ei, sem_stage):
    sid = lax.axis_index("s")
    wid = sid * NC + lax.axis_index("c")
    t0 = wid * N_TILES // NW
    t1 = (wid + 1) * N_TILES // NW
    base = t0 * TILE
    n_w = (t1 - t0) * TILE

    cp_ei = pltpu.async_copy(ei_hbm.at[:, pl.ds(base, n_w)],
                             ei_v.at[:, pl.ds(0, n_w)], sem_ei)
    @pl.when(sid == 0)
    def _():
        pltpu.async_copy(y_hbm.at[0], y_sh, sem_stage).wait()

    plsc.subcore_barrier()
    cp_y = pltpu.async_copy(y_sh, y_v, sem_y)
    cp_y.wait()
    cp_ei.wait()

    dst_off = jnp.full((LANES,), PADN, jnp.int32)

    @plsc.parallel_loop(0, n_w, LANES, unroll=8)
    def _(off):
        s_idx = ei_v[0, pl.ds(off, LANES)]
        d_idx = ei_v[1, pl.ds(off, LANES)]
        g1 = plsc.load_gather(y_v, [s_idx])
        g2 = plsc.load_gather(y_v, [d_idx + dst_off])
        out_v[pl.ds(off, LANES)] = g1 + g2

    pltpu.sync_copy(out_v.at[pl.ds(0, n_w)], out_hbm.at[0, pl.ds(base, n_w)])


_edge_call = pl.kernel(
    _edge_body,
    out_type=jax.ShapeDtypeStruct((1, N_EDGES), jnp.float32),
    mesh=plsc.VectorSubcoreMesh(core_axis_name="c", subcore_axis_name="s"),
    compiler_params=pltpu.CompilerParams(needs_layout_passes=False),
    scratch_types=[
        pltpu.VMEM_SHARED((2 * PADN,), jnp.float32),
        pltpu.VMEM((2 * PADN,), jnp.float32),
        pltpu.VMEM((2, E_MAX), jnp.int32),
        pltpu.VMEM((E_MAX,), jnp.float32),
        pltpu.SemaphoreType.DMA,
        pltpu.SemaphoreType.DMA,
        pltpu.SemaphoreType.DMA,
    ],
)


def kernel(X, edge_index, W1, W2):
    Y = _node_tables(X, W1, W2)
    out = _edge_call(Y, edge_index.astype(jnp.int32))
    return out.reshape(N_EDGES, 1)


# speedup vs baseline: 1.0254x; 1.0254x over previous
import jax
import jax.numpy as jnp
from jax import lax
from jax.experimental import pallas as pl
from jax.experimental.pallas import tpu as pltpu
from jax.experimental.pallas import tpu_sc as plsc

N_NODES = 10000
N_EDGES = 320000
NC = 2
NS = 16
NW = NC * NS
LANES = 16
TILE = 128
N_TILES = N_EDGES // TILE
E_MAX = (N_TILES // NW + 1) * TILE
PADN = 10240


def _matmul_body(x_ref, w1t_ref, w2t_ref, y_ref):
    wt = jnp.concatenate([w1t_ref[...], w2t_ref[...]], axis=0)
    y2d = lax.dot_general(
        wt, x_ref[...], dimension_numbers=(((1,), (1,)), ((), ())),
        preferred_element_type=jnp.float32)
    y_ref[0:1, pl.ds(0, N_NODES)] = y2d[0:1, :]
    y_ref[0:1, pl.ds(PADN, N_NODES)] = y2d[1:2, :]


def _node_tables(X, W1, W2):
    return pl.pallas_call(
        _matmul_body,
        out_shape=jax.ShapeDtypeStruct((1, 2 * PADN), jnp.float32),
    )(X, W1.T, W2.T)


def _edge_body(y_hbm, ei_hbm, out_hbm, y_sh, y_v, ei_v, out_v,
               sem_y, sem_ei, sem_stage):
    sid = lax.axis_index("s")
    wid = sid * NC + lax.axis_index("c")
    t0 = wid * N_TILES // NW
    t1 = (wid + 1) * N_TILES // NW
    base = t0 * TILE
    n_w = (t1 - t0) * TILE

    cp_ei = pltpu.async_copy(ei_hbm.at[:, pl.ds(base, n_w)],
                             ei_v.at[:, pl.ds(0, n_w)], sem_ei)
    @pl.when(sid == 0)
    def _():
        pltpu.async_copy(y_hbm.at[0], y_sh, sem_stage).wait()

    plsc.subcore_barrier()
    cp_y = pltpu.async_copy(y_sh, y_v, sem_y)
    cp_y.wait()
    cp_ei.wait()

    dst_off = jnp.full((LANES,), PADN, jnp.int32)

    @plsc.parallel_loop(0, n_w, LANES, unroll=8)
    def _(off):
        s_idx = ei_v[0, pl.ds(off, LANES)]
        d_idx = ei_v[1, pl.ds(off, LANES)]
        g1 = plsc.load_gather(y_v, [s_idx])
        g2 = plsc.load_gather(y_v, [d_idx + dst_off])
        out_v[pl.ds(off, LANES)] = g1 + g2

    pltpu.sync_copy(out_v.at[pl.ds(0, n_w)], out_hbm.at[0, pl.ds(base, n_w)])


_edge_call = pl.kernel(
    _edge_body,
    out_type=jax.ShapeDtypeStruct((1, N_EDGES), jnp.float32),
    mesh=plsc.VectorSubcoreMesh(core_axis_name="c", subcore_axis_name="s"),
    compiler_params=pltpu.CompilerParams(needs_layout_passes=False),
    scratch_types=[
        pltpu.VMEM_SHARED((2 * PADN,), jnp.float32),
        pltpu.VMEM((2 * PADN,), jnp.float32),
        pltpu.VMEM((2, E_MAX), jnp.int32),
        pltpu.VMEM((E_MAX,), jnp.float32),
        pltpu.SemaphoreType.DMA,
        pltpu.SemaphoreType.DMA,
        pltpu.SemaphoreType.DMA,
    ],
)


def kernel(X, edge_index, W1, W2):
    Y = _node_tables(X, W1, W2)
    out = _edge_call(Y, edge_index.astype(jnp.int32))
    return out.reshape(N_EDGES, 1)
